# baseline (speedup 1.0000x reference)
import functools

import jax
import jax.numpy as jnp
from jax import lax
from jax.experimental import pallas as pl
from jax.experimental.pallas import tpu as pltpu
from jax.experimental.pallas import tpu_sc as plsc

N = 10000
E = 320000
D = 128
DO = 64
NP = 10240
NC = 2
NS = 16
NW = NC * NS
W = 128
SLAB = 2
J2 = 40
J = SLAB * J2
EP = NW * J * W
SL = NP // NS
RB = 10

_MESH = dict(core_axis_name="c", subcore_axis_name="s",
             num_cores=NC, num_subcores=NS)


def _deg_body(ew_hbm, cnt_hbm, dst_v, ones_v, cnt_v, acc, sem):
    c = lax.axis_index("c")
    s = lax.axis_index("s")
    w = s * NC + c

    z16 = jnp.zeros((16,), jnp.float32)

    def zb(i, carry):
        cnt_v[pl.ds(i * 16, 16)] = z16
        return carry

    lax.fori_loop(0, SL // 16, zb, 0)
    pltpu.sync_copy(cnt_v, acc.at[pl.ds(s * SL, SL)])
    o16 = jnp.ones((16,), jnp.float32)
    for k in range(W // 16):
        ones_v[pl.ds(k * 16, 16)] = o16
    plsc.subcore_barrier()

    for h in range(SLAB):
        pltpu.sync_copy(ew_hbm.at[1, w].at[pl.ds(h * J2, J2)], dst_v)

        def grp(g, carry):
            j = g * 2
            for k in range(2):
                pltpu.async_copy(ones_v, acc.at[dst_v.at[j + k]], sem, add=True)
            for k in range(2):
                pltpu.make_async_copy(ones_v, acc.at[dst_v.at[j + k]], sem).wait()
            return carry

        lax.fori_loop(0, J2 // 2, grp, 0)

    plsc.subcore_barrier()
    pltpu.sync_copy(acc.at[pl.ds(s * SL, SL)],
                    cnt_hbm.at[c].at[pl.ds(s * SL, SL)])


_deg_dis = functools.partial(
    pl.kernel,
    out_type=jax.ShapeDtypeStruct((NC, NP), jnp.float32),
    mesh=plsc.VectorSubcoreMesh(**_MESH),
    scratch_types=[
        pltpu.VMEM((J2, W), jnp.int32),
        pltpu.VMEM((W,), jnp.float32),
        pltpu.VMEM((SL,), jnp.float32),
        pltpu.VMEM_SHARED((NP,), jnp.float32),
        pltpu.SemaphoreType.DMA,
    ],
)(_deg_body)


def _agg_body(g_hbm, ew_hbm, zeros_hbm, out_hbm,
              src_v, dst_v, rows0, rows1, acc, gsem0, gsem1):
    c = lax.axis_index("c")
    s = lax.axis_index("s")
    w = s * NC + c

    pltpu.sync_copy(zeros_hbm, rows0)
    for t in range(SL // W):
        pltpu.sync_copy(rows0, acc.at[pl.ds(s * SL + t * W, W)])
    plsc.subcore_barrier()

    for h in range(SLAB):
        pltpu.sync_copy(ew_hbm.at[0, w].at[pl.ds(h * J2, J2)], src_v)
        pltpu.sync_copy(ew_hbm.at[1, w].at[pl.ds(h * J2, J2)], dst_v)

        pltpu.async_copy(g_hbm.at[src_v.at[0]], rows0, gsem0)
        pltpu.async_copy(g_hbm.at[src_v.at[1]], rows1, gsem1)

        def body(jj, carry):
            j0 = 2 * jj
            pltpu.make_async_copy(g_hbm.at[src_v.at[j0]], rows0, gsem0).wait()
            pltpu.sync_copy(rows0, acc.at[dst_v.at[j0]], add=True)

            @pl.when(jj < J2 // 2 - 1)
            def _():
                pltpu.async_copy(g_hbm.at[src_v.at[j0 + 2]], rows0, gsem0)

            j1 = j0 + 1
            pltpu.make_async_copy(g_hbm.at[src_v.at[j1]], rows1, gsem1).wait()
            pltpu.sync_copy(rows1, acc.at[dst_v.at[j1]], add=True)

            @pl.when(jj < J2 // 2 - 1)
            def _():
                pltpu.async_copy(g_hbm.at[src_v.at[j1 + 2]], rows1, gsem1)

            return carry

        lax.fori_loop(0, J2 // 2, body, 0)

    plsc.subcore_barrier()
    pltpu.sync_copy(acc.at[pl.ds(s * SL, SL)], out_hbm.at[c].at[pl.ds(s * SL, SL)])


_agg = functools.partial(
    pl.kernel,
    out_type=jax.ShapeDtypeStruct((NC, NP, D), jnp.float32),
    mesh=plsc.VectorSubcoreMesh(**_MESH),
    scratch_types=[
        pltpu.VMEM((J2, W), jnp.int32),
        pltpu.VMEM((J2, W), jnp.int32),
        pltpu.VMEM((W, D), jnp.float32),
        pltpu.VMEM((W, D), jnp.float32),
        pltpu.VMEM_SHARED((NP, D), jnp.float32),
        pltpu.SemaphoreType.DMA,
        pltpu.SemaphoreType.DMA,
    ],
)(_agg_body)


def _tc0_body(x_ref, w_ref, cnt_ref, o_ref, dis_ref):
    dis = lax.rsqrt(cnt_ref[0] + cnt_ref[1] + 1.0)
    dis_ref[...] = dis
    o_ref[...] = jnp.dot(x_ref[...], w_ref[...],
                         preferred_element_type=jnp.float32) * dis


def _tcmid_body(p_ref, g_ref, dis_ref, b_ref, w_ref, o_ref):
    h = (p_ref[0] + p_ref[1] + g_ref[...]) * dis_ref[...] + b_ref[...]
    h = jnp.maximum(h, 0.0)
    o_ref[...] = jnp.dot(h, w_ref[...],
                         preferred_element_type=jnp.float32) * dis_ref[...]


def _tcfin_body(p_ref, g_ref, dis_ref, bmu_ref, bls_ref, o_ref):
    o = (p_ref[0] + p_ref[1] + g_ref[...]) * dis_ref[...]
    o_ref[0] = o[:, :DO] + bmu_ref[...]
    o_ref[1] = o[:, DO:] + bls_ref[...]


_BLK = 1000


def _tc0(x, w, cnt):
    return pl.pallas_call(
        _tc0_body,
        grid=(RB,),
        in_specs=[
            pl.BlockSpec((_BLK, D), lambda i: (i, 0)),
            pl.BlockSpec((D, D), lambda i: (0, 0)),
            pl.BlockSpec((NC, _BLK, 1), lambda i: (0, i, 0)),
        ],
        out_specs=[
            pl.BlockSpec((_BLK, D), lambda i: (i, 0)),
            pl.BlockSpec((_BLK, 1), lambda i: (i, 0)),
        ],
        out_shape=[
            jax.ShapeDtypeStruct((N, D), jnp.float32),
            jax.ShapeDtypeStruct((N, 1), jnp.float32),
        ],
    )(x, w, cnt)


def _tcmid(p, g, dis, b, w):
    return pl.pallas_call(
        _tcmid_body,
        grid=(RB,),
        in_specs=[
            pl.BlockSpec((NC, _BLK, D), lambda i: (0, i, 0)),
            pl.BlockSpec((_BLK, D), lambda i: (i, 0)),
            pl.BlockSpec((_BLK, 1), lambda i: (i, 0)),
            pl.BlockSpec((1, D), lambda i: (0, 0)),
            pl.BlockSpec((D, D), lambda i: (0, 0)),
        ],
        out_specs=pl.BlockSpec((_BLK, D), lambda i: (i, 0)),
        out_shape=jax.ShapeDtypeStruct((N, D), jnp.float32),
    )(p, g, dis, b, w)


def _tcfin(p, g, dis, bmu, bls):
    return pl.pallas_call(
        _tcfin_body,
        grid=(RB,),
        in_specs=[
            pl.BlockSpec((NC, _BLK, D), lambda i: (0, i, 0)),
            pl.BlockSpec((_BLK, D), lambda i: (i, 0)),
            pl.BlockSpec((_BLK, 1), lambda i: (i, 0)),
            pl.BlockSpec((1, DO), lambda i: (0, 0)),
            pl.BlockSpec((1, DO), lambda i: (0, 0)),
        ],
        out_specs=pl.BlockSpec((2, _BLK, DO), lambda i: (0, i, 0)),
        out_shape=jax.ShapeDtypeStruct((2, N, DO), jnp.float32),
    )(p, g, dis, bmu, bls)


def kernel(x, edge_index, W1, b1, W2, b2, W_mu, b_mu, W_ls, b_ls):
    ar = jnp.arange(EP - E, dtype=jnp.int32)
    padb = jnp.stack([ar % N, N + ar % (NP - N)])
    ew = jnp.concatenate([edge_index, padb], axis=1).reshape(2, NW, J, W)
    zeros = jnp.zeros((W, D), jnp.float32)
    Wml = jnp.concatenate([W_mu, W_ls], axis=1)

    cnt = _deg_dis(ew)
    g1, dis_c = _tc0(x, W1, cnt.reshape(NC, NP, 1))
    p1 = _agg(g1, ew, zeros)
    g2 = _tcmid(p1, g1, dis_c, b1.reshape(1, D), W2)
    p2 = _agg(g2, ew, zeros)
    g3 = _tcmid(p2, g2, dis_c, b2.reshape(1, D), Wml)
    p3 = _agg(g3, ew, zeros)
    out = _tcfin(p3, g3, dis_c, b_mu.reshape(1, DO), b_ls.reshape(1, DO))
    return (out[0], out[1])


# speedup vs baseline: 1.0174x; 1.0174x over previous
import functools

import jax
import jax.numpy as jnp
from jax import lax
from jax.experimental import pallas as pl
from jax.experimental.pallas import tpu as pltpu
from jax.experimental.pallas import tpu_sc as plsc

N = 10000
E = 320000
D = 128
DO = 64
NP = 10240
NC = 2
NS = 16
NW = NC * NS
W = 128
SLAB = 2
J2 = 40
J = SLAB * J2
EP = NW * J * W
SL = NP // NS
RB = 10

_MESH = dict(core_axis_name="c", subcore_axis_name="s",
             num_cores=NC, num_subcores=NS)


def _deg_body(ew_hbm, cnt_hbm, dst_v, ones_v, cnt_v, acc, sem):
    c = lax.axis_index("c")
    s = lax.axis_index("s")
    w = s * NC + c

    z16 = jnp.zeros((16,), jnp.float32)

    def zb(i, carry):
        cnt_v[pl.ds(i * 16, 16)] = z16
        return carry

    lax.fori_loop(0, SL // 16, zb, 0)
    pltpu.sync_copy(cnt_v, acc.at[pl.ds(s * SL, SL)])
    o16 = jnp.ones((16,), jnp.float32)
    for k in range(W // 16):
        ones_v[pl.ds(k * 16, 16)] = o16
    plsc.subcore_barrier()

    for h in range(SLAB):
        pltpu.sync_copy(ew_hbm.at[1, w].at[pl.ds(h * J2, J2)], dst_v)

        def grp(g, carry):
            j = g * 2
            for k in range(2):
                pltpu.async_copy(ones_v, acc.at[dst_v.at[j + k]], sem, add=True)
            for k in range(2):
                pltpu.make_async_copy(ones_v, acc.at[dst_v.at[j + k]], sem).wait()
            return carry

        lax.fori_loop(0, J2 // 2, grp, 0)

    plsc.subcore_barrier()
    pltpu.sync_copy(acc.at[pl.ds(s * SL, SL)],
                    cnt_hbm.at[c].at[pl.ds(s * SL, SL)])


_deg_dis = functools.partial(
    pl.kernel,
    out_type=jax.ShapeDtypeStruct((NC, NP), jnp.float32),
    mesh=plsc.VectorSubcoreMesh(**_MESH),
    scratch_types=[
        pltpu.VMEM((J2, W), jnp.int32),
        pltpu.VMEM((W,), jnp.float32),
        pltpu.VMEM((SL,), jnp.float32),
        pltpu.VMEM_SHARED((NP,), jnp.float32),
        pltpu.SemaphoreType.DMA,
    ],
)(_deg_body)


def _agg_body(g_hbm, ew_hbm, zeros_hbm, out_hbm,
              src_v, dst_v, rows0, rows1, acc, gsem0, gsem1):
    c = lax.axis_index("c")
    s = lax.axis_index("s")
    w = s * NC + c

    pltpu.sync_copy(ew_hbm.at[0, w].at[pl.ds(0, J2)], src_v)
    pltpu.sync_copy(ew_hbm.at[1, w].at[pl.ds(0, J2)], dst_v)
    pltpu.async_copy(g_hbm.at[src_v.at[0]], rows0, gsem0)
    pltpu.async_copy(g_hbm.at[src_v.at[1]], rows1, gsem1)

    @pl.when((c == 0) & (s < NS - 1))
    def _():
        pltpu.sync_copy(g_hbm.at[pl.ds(s * SL, SL)], acc.at[pl.ds(s * SL, SL)])

    @pl.when((c == 0) & (s == NS - 1))
    def _():
        pltpu.sync_copy(g_hbm.at[pl.ds((NS - 1) * SL, N - (NS - 1) * SL)],
                        acc.at[pl.ds((NS - 1) * SL, N - (NS - 1) * SL)])
        pltpu.sync_copy(zeros_hbm.at[pl.ds(N, NP - N)],
                        acc.at[pl.ds(N, NP - N)])

    @pl.when(c == 1)
    def _():
        pltpu.sync_copy(zeros_hbm.at[pl.ds(s * SL, SL)],
                        acc.at[pl.ds(s * SL, SL)])

    plsc.subcore_barrier()

    for h in range(SLAB):
        if h > 0:
            pltpu.sync_copy(ew_hbm.at[0, w].at[pl.ds(h * J2, J2)], src_v)
            pltpu.sync_copy(ew_hbm.at[1, w].at[pl.ds(h * J2, J2)], dst_v)
            pltpu.async_copy(g_hbm.at[src_v.at[0]], rows0, gsem0)
            pltpu.async_copy(g_hbm.at[src_v.at[1]], rows1, gsem1)

        def body(jj, carry):
            j0 = 2 * jj
            pltpu.make_async_copy(g_hbm.at[src_v.at[j0]], rows0, gsem0).wait()
            pltpu.sync_copy(rows0, acc.at[dst_v.at[j0]], add=True)

            @pl.when(jj < J2 // 2 - 1)
            def _():
                pltpu.async_copy(g_hbm.at[src_v.at[j0 + 2]], rows0, gsem0)

            j1 = j0 + 1
            pltpu.make_async_copy(g_hbm.at[src_v.at[j1]], rows1, gsem1).wait()
            pltpu.sync_copy(rows1, acc.at[dst_v.at[j1]], add=True)

            @pl.when(jj < J2 // 2 - 1)
            def _():
                pltpu.async_copy(g_hbm.at[src_v.at[j1 + 2]], rows1, gsem1)

            return carry

        lax.fori_loop(0, J2 // 2, body, 0)

    plsc.subcore_barrier()
    pltpu.sync_copy(acc.at[pl.ds(s * SL, SL)], out_hbm.at[c].at[pl.ds(s * SL, SL)])


_agg = functools.partial(
    pl.kernel,
    out_type=jax.ShapeDtypeStruct((NC, NP, D), jnp.float32),
    mesh=plsc.VectorSubcoreMesh(**_MESH),
    scratch_types=[
        pltpu.VMEM((J2, W), jnp.int32),
        pltpu.VMEM((J2, W), jnp.int32),
        pltpu.VMEM((W, D), jnp.float32),
        pltpu.VMEM((W, D), jnp.float32),
        pltpu.VMEM_SHARED((NP, D), jnp.float32),
        pltpu.SemaphoreType.DMA,
        pltpu.SemaphoreType.DMA,
    ],
)(_agg_body)


def _tc0_body(x_ref, w_ref, cnt_ref, o_ref, dis_ref):
    dis = lax.rsqrt(cnt_ref[0] + cnt_ref[1] + 1.0)
    dis_ref[...] = dis
    o_ref[...] = jnp.dot(x_ref[...], w_ref[...],
                         preferred_element_type=jnp.float32) * dis


def _tcmid_body(p_ref, dis_ref, b_ref, w_ref, o_ref):
    h = (p_ref[0] + p_ref[1]) * dis_ref[...] + b_ref[...]
    h = jnp.maximum(h, 0.0)
    o_ref[...] = jnp.dot(h, w_ref[...],
                         preferred_element_type=jnp.float32) * dis_ref[...]


def _tcfin_body(p_ref, dis_ref, bmu_ref, bls_ref, o_ref):
    o = (p_ref[0] + p_ref[1]) * dis_ref[...]
    o_ref[0] = o[:, :DO] + bmu_ref[...]
    o_ref[1] = o[:, DO:] + bls_ref[...]


_BLK = 1000


def _tc0(x, w, cnt):
    return pl.pallas_call(
        _tc0_body,
        grid=(RB,),
        in_specs=[
            pl.BlockSpec((_BLK, D), lambda i: (i, 0)),
            pl.BlockSpec((D, D), lambda i: (0, 0)),
            pl.BlockSpec((NC, _BLK, 1), lambda i: (0, i, 0)),
        ],
        out_specs=[
            pl.BlockSpec((_BLK, D), lambda i: (i, 0)),
            pl.BlockSpec((_BLK, 1), lambda i: (i, 0)),
        ],
        out_shape=[
            jax.ShapeDtypeStruct((N, D), jnp.float32),
            jax.ShapeDtypeStruct((N, 1), jnp.float32),
        ],
    )(x, w, cnt)


def _tcmid(p, dis, b, w):
    return pl.pallas_call(
        _tcmid_body,
        grid=(RB,),
        in_specs=[
            pl.BlockSpec((NC, _BLK, D), lambda i: (0, i, 0)),
            pl.BlockSpec((_BLK, 1), lambda i: (i, 0)),
            pl.BlockSpec((1, D), lambda i: (0, 0)),
            pl.BlockSpec((D, D), lambda i: (0, 0)),
        ],
        out_specs=pl.BlockSpec((_BLK, D), lambda i: (i, 0)),
        out_shape=jax.ShapeDtypeStruct((N, D), jnp.float32),
    )(p, dis, b, w)


def _tcfin(p, dis, bmu, bls):
    return pl.pallas_call(
        _tcfin_body,
        grid=(RB,),
        in_specs=[
            pl.BlockSpec((NC, _BLK, D), lambda i: (0, i, 0)),
            pl.BlockSpec((_BLK, 1), lambda i: (i, 0)),
            pl.BlockSpec((1, DO), lambda i: (0, 0)),
            pl.BlockSpec((1, DO), lambda i: (0, 0)),
        ],
        out_specs=pl.BlockSpec((2, _BLK, DO), lambda i: (0, i, 0)),
        out_shape=jax.ShapeDtypeStruct((2, N, DO), jnp.float32),
    )(p, dis, bmu, bls)


def kernel(x, edge_index, W1, b1, W2, b2, W_mu, b_mu, W_ls, b_ls):
    ar = jnp.arange(EP - E, dtype=jnp.int32)
    padb = jnp.stack([ar % N, N + ar % (NP - N)])
    ew = jnp.concatenate([edge_index, padb], axis=1).reshape(2, NW, J, W)
    zeros = jnp.zeros((NP, D), jnp.float32)
    Wml = jnp.concatenate([W_mu, W_ls], axis=1)

    cnt = _deg_dis(ew)
    g1, dis_c = _tc0(x, W1, cnt.reshape(NC, NP, 1))
    p1 = _agg(g1, ew, zeros)
    g2 = _tcmid(p1, dis_c, b1.reshape(1, D), W2)
    p2 = _agg(g2, ew, zeros)
    g3 = _tcmid(p2, dis_c, b2.reshape(1, D), Wml)
    p3 = _agg(g3, ew, zeros)
    out = _tcfin(p3, dis_c, b_mu.reshape(1, DO), b_ls.reshape(1, DO))
    return (out[0], out[1])


# speedup vs baseline: 1.0201x; 1.0026x over previous
import functools

import jax
import jax.numpy as jnp
from jax import lax
from jax.experimental import pallas as pl
from jax.experimental.pallas import tpu as pltpu
from jax.experimental.pallas import tpu_sc as plsc

N = 10000
E = 320000
D = 128
DO = 64
NP = 10240
NC = 2
NS = 16
NW = NC * NS
W = 128
SLAB = 2
J2 = 40
J = SLAB * J2
EP = NW * J * W
SL = NP // NS
RB = 10

_MESH = dict(core_axis_name="c", subcore_axis_name="s",
             num_cores=NC, num_subcores=NS)


def _deg_body(ew_hbm, cnt_hbm, dst_v, ones_v, cnt_v, acc, sem):
    c = lax.axis_index("c")
    s = lax.axis_index("s")
    w = s * NC + c

    z16 = jnp.zeros((16,), jnp.float32)

    def zb(i, carry):
        cnt_v[pl.ds(i * 16, 16)] = z16
        return carry

    lax.fori_loop(0, SL // 16, zb, 0)
    pltpu.sync_copy(cnt_v, acc.at[pl.ds(s * SL, SL)])
    o16 = jnp.ones((16,), jnp.float32)
    for k in range(W // 16):
        ones_v[pl.ds(k * 16, 16)] = o16
    plsc.subcore_barrier()

    for h in range(SLAB):
        pltpu.sync_copy(ew_hbm.at[1, w].at[pl.ds(h * J2, J2)], dst_v)

        def grp(g, carry):
            j = g * 4
            for k in range(4):
                pltpu.async_copy(ones_v, acc.at[dst_v.at[j + k]], sem, add=True)
            for k in range(4):
                pltpu.make_async_copy(ones_v, acc.at[dst_v.at[j + k]], sem).wait()
            return carry

        lax.fori_loop(0, J2 // 4, grp, 0)

    plsc.subcore_barrier()
    pltpu.sync_copy(acc.at[pl.ds(s * SL, SL)],
                    cnt_hbm.at[c].at[pl.ds(s * SL, SL)])


_deg_dis = functools.partial(
    pl.kernel,
    out_type=jax.ShapeDtypeStruct((NC, NP), jnp.float32),
    mesh=plsc.VectorSubcoreMesh(**_MESH),
    scratch_types=[
        pltpu.VMEM((J2, W), jnp.int32),
        pltpu.VMEM((W,), jnp.float32),
        pltpu.VMEM((SL,), jnp.float32),
        pltpu.VMEM_SHARED((NP,), jnp.float32),
        pltpu.SemaphoreType.DMA,
    ],
)(_deg_body)


def _agg_body(g_hbm, ew_hbm, zeros_hbm, out_hbm,
              src_v, dst_v, rows0, rows1, acc, gsem0, gsem1):
    c = lax.axis_index("c")
    s = lax.axis_index("s")
    w = s * NC + c

    pltpu.sync_copy(ew_hbm.at[0, w].at[pl.ds(0, J2)], src_v)
    pltpu.sync_copy(ew_hbm.at[1, w].at[pl.ds(0, J2)], dst_v)
    pltpu.async_copy(g_hbm.at[src_v.at[0]], rows0, gsem0)
    pltpu.async_copy(g_hbm.at[src_v.at[1]], rows1, gsem1)

    @pl.when((c == 0) & (s < NS - 1))
    def _():
        pltpu.sync_copy(g_hbm.at[pl.ds(s * SL, SL)], acc.at[pl.ds(s * SL, SL)])

    @pl.when((c == 0) & (s == NS - 1))
    def _():
        pltpu.sync_copy(g_hbm.at[pl.ds((NS - 1) * SL, N - (NS - 1) * SL)],
                        acc.at[pl.ds((NS - 1) * SL, N - (NS - 1) * SL)])
        pltpu.sync_copy(zeros_hbm.at[pl.ds(N, NP - N)],
                        acc.at[pl.ds(N, NP - N)])

    @pl.when(c == 1)
    def _():
        pltpu.sync_copy(zeros_hbm.at[pl.ds(s * SL, SL)],
                        acc.at[pl.ds(s * SL, SL)])

    plsc.subcore_barrier()

    for h in range(SLAB):
        if h > 0:
            pltpu.sync_copy(ew_hbm.at[0, w].at[pl.ds(h * J2, J2)], src_v)
            pltpu.sync_copy(ew_hbm.at[1, w].at[pl.ds(h * J2, J2)], dst_v)
            pltpu.async_copy(g_hbm.at[src_v.at[0]], rows0, gsem0)
            pltpu.async_copy(g_hbm.at[src_v.at[1]], rows1, gsem1)

        def body(jj, carry):
            j0 = 2 * jj
            pltpu.make_async_copy(g_hbm.at[src_v.at[j0]], rows0, gsem0).wait()
            pltpu.sync_copy(rows0, acc.at[dst_v.at[j0]], add=True)

            @pl.when(jj < J2 // 2 - 1)
            def _():
                pltpu.async_copy(g_hbm.at[src_v.at[j0 + 2]], rows0, gsem0)

            j1 = j0 + 1
            pltpu.make_async_copy(g_hbm.at[src_v.at[j1]], rows1, gsem1).wait()
            pltpu.sync_copy(rows1, acc.at[dst_v.at[j1]], add=True)

            @pl.when(jj < J2 // 2 - 1)
            def _():
                pltpu.async_copy(g_hbm.at[src_v.at[j1 + 2]], rows1, gsem1)

            return carry

        lax.fori_loop(0, J2 // 2, body, 0)

    plsc.subcore_barrier()
    pltpu.sync_copy(acc.at[pl.ds(s * SL, SL)], out_hbm.at[c].at[pl.ds(s * SL, SL)])


_agg = functools.partial(
    pl.kernel,
    out_type=jax.ShapeDtypeStruct((NC, NP, D), jnp.float32),
    mesh=plsc.VectorSubcoreMesh(**_MESH),
    scratch_types=[
        pltpu.VMEM((J2, W), jnp.int32),
        pltpu.VMEM((J2, W), jnp.int32),
        pltpu.VMEM((W, D), jnp.float32),
        pltpu.VMEM((W, D), jnp.float32),
        pltpu.VMEM_SHARED((NP, D), jnp.float32),
        pltpu.SemaphoreType.DMA,
        pltpu.SemaphoreType.DMA,
    ],
)(_agg_body)


def _tc0_body(x_ref, w_ref, cnt_ref, o_ref, dis_ref):
    dis = lax.rsqrt(cnt_ref[0] + cnt_ref[1] + 1.0)
    dis_ref[...] = dis
    o_ref[...] = jnp.dot(x_ref[...], w_ref[...],
                         preferred_element_type=jnp.float32) * dis


def _tcmid_body(p_ref, dis_ref, b_ref, w_ref, o_ref):
    h = (p_ref[0] + p_ref[1]) * dis_ref[...] + b_ref[...]
    h = jnp.maximum(h, 0.0)
    o_ref[...] = jnp.dot(h, w_ref[...],
                         preferred_element_type=jnp.float32) * dis_ref[...]


def _tcfin_body(p_ref, dis_ref, bmu_ref, bls_ref, o_ref):
    o = (p_ref[0] + p_ref[1]) * dis_ref[...]
    o_ref[0] = o[:, :DO] + bmu_ref[...]
    o_ref[1] = o[:, DO:] + bls_ref[...]


_BLK = 1000


def _tc0(x, w, cnt):
    return pl.pallas_call(
        _tc0_body,
        grid=(RB,),
        in_specs=[
            pl.BlockSpec((_BLK, D), lambda i: (i, 0)),
            pl.BlockSpec((D, D), lambda i: (0, 0)),
            pl.BlockSpec((NC, _BLK, 1), lambda i: (0, i, 0)),
        ],
        out_specs=[
            pl.BlockSpec((_BLK, D), lambda i: (i, 0)),
            pl.BlockSpec((_BLK, 1), lambda i: (i, 0)),
        ],
        out_shape=[
            jax.ShapeDtypeStruct((N, D), jnp.float32),
            jax.ShapeDtypeStruct((N, 1), jnp.float32),
        ],
    )(x, w, cnt)


def _tcmid(p, dis, b, w):
    return pl.pallas_call(
        _tcmid_body,
        grid=(RB,),
        in_specs=[
            pl.BlockSpec((NC, _BLK, D), lambda i: (0, i, 0)),
            pl.BlockSpec((_BLK, 1), lambda i: (i, 0)),
            pl.BlockSpec((1, D), lambda i: (0, 0)),
            pl.BlockSpec((D, D), lambda i: (0, 0)),
        ],
        out_specs=pl.BlockSpec((_BLK, D), lambda i: (i, 0)),
        out_shape=jax.ShapeDtypeStruct((N, D), jnp.float32),
    )(p, dis, b, w)


def _tcfin(p, dis, bmu, bls):
    return pl.pallas_call(
        _tcfin_body,
        grid=(RB,),
        in_specs=[
            pl.BlockSpec((NC, _BLK, D), lambda i: (0, i, 0)),
            pl.BlockSpec((_BLK, 1), lambda i: (i, 0)),
            pl.BlockSpec((1, DO), lambda i: (0, 0)),
            pl.BlockSpec((1, DO), lambda i: (0, 0)),
        ],
        out_specs=pl.BlockSpec((2, _BLK, DO), lambda i: (0, i, 0)),
        out_shape=jax.ShapeDtypeStruct((2, N, DO), jnp.float32),
    )(p, dis, bmu, bls)


def kernel(x, edge_index, W1, b1, W2, b2, W_mu, b_mu, W_ls, b_ls):
    ar = jnp.arange(EP - E, dtype=jnp.int32)
    padb = jnp.stack([ar % N, N + ar % (NP - N)])
    ew = jnp.concatenate([edge_index, padb], axis=1).reshape(2, NW, J, W)
    zeros = jnp.zeros((NP, D), jnp.float32)
    Wml = jnp.concatenate([W_mu, W_ls], axis=1)

    cnt = _deg_dis(ew)
    g1, dis_c = _tc0(x, W1, cnt.reshape(NC, NP, 1))
    p1 = _agg(g1, ew, zeros)
    g2 = _tcmid(p1, dis_c, b1.reshape(1, D), W2)
    p2 = _agg(g2, ew, zeros)
    g3 = _tcmid(p2, dis_c, b2.reshape(1, D), Wml)
    p3 = _agg(g3, ew, zeros)
    out = _tcfin(p3, dis_c, b_mu.reshape(1, DO), b_ls.reshape(1, DO))
    return (out[0], out[1])
